# Initial kernel scaffold; baseline (speedup 1.0000x reference)
import jax
import jax.numpy as jnp
from jax.experimental import pallas as pl


def kernel(x, W_embed):
    raise NotImplementedError("write your pallas kernel here")


# speedup vs baseline: 1.8447x; 1.8447x over previous
import functools

import jax
import jax.numpy as jnp
from jax import lax
from jax.experimental import pallas as pl
from jax.experimental.pallas import tpu as pltpu
from jax.experimental.pallas import tpu_sc as plsc

EMBED = 64

_info = plsc.get_sparse_core_info()
_NC, _NS = _info.num_cores, _info.num_subcores
_NW = _NC * _NS

CHUNK = 1024


@functools.partial(jax.jit, static_argnums=())
def _embedding_gather(W_embed, flat_idx):
    B = flat_idx.shape[0]
    assert B % (8 * _NW) == 0
    b_per_w = B // _NW
    n_chunks = b_per_w // CHUNK
    assert n_chunks * CHUNK == b_per_w

    mesh = plsc.VectorSubcoreMesh(core_axis_name="c", subcore_axis_name="s")

    @functools.partial(
        pl.kernel,
        mesh=mesh,
        out_type=jax.ShapeDtypeStruct((B, EMBED), jnp.float32),
        scratch_types=[
            pltpu.VMEM((CHUNK,), jnp.int32),
            pltpu.VMEM((CHUNK, EMBED), jnp.float32),
            pltpu.SemaphoreType.DMA,
        ],
        compiler_params=pltpu.CompilerParams(use_tc_tiling_on_sc=False),
    )
    def k(table_hbm, idx_hbm, out_hbm, idx_v, rows_v, sem):
        wid = lax.axis_index("s") * _NC + lax.axis_index("c")
        base = wid * b_per_w

        def chunk_body(g, carry):
            off = base + g * CHUNK
            pltpu.sync_copy(idx_hbm.at[pl.ds(off, CHUNK)], idx_v)
            pltpu.async_copy(table_hbm.at[idx_v], rows_v, sem).wait()
            pltpu.sync_copy(rows_v, out_hbm.at[pl.ds(off, CHUNK)])
            return carry

        lax.fori_loop(0, n_chunks, chunk_body, 0)

    return k(W_embed, flat_idx)


def kernel(x, W_embed):
    flat = x.reshape(-1).astype(jnp.int32)
    out = _embedding_gather(W_embed, flat)
    return out.reshape(x.shape + (EMBED,))


# speedup vs baseline: 1.8762x; 1.0171x over previous
import functools

import jax
import jax.numpy as jnp
from jax import lax
from jax.experimental import pallas as pl
from jax.experimental.pallas import tpu as pltpu
from jax.experimental.pallas import tpu_sc as plsc

EMBED = 64

_info = plsc.get_sparse_core_info()
_NC, _NS = _info.num_cores, _info.num_subcores
_NW = _NC * _NS

CHUNK = 400
NBUF = 4


@jax.jit
def _embedding_gather(W_embed, flat_idx):
    B = flat_idx.shape[0]
    assert B % (8 * _NW) == 0
    b_per_w = B // _NW
    n_chunks = b_per_w // CHUNK
    assert n_chunks * CHUNK == b_per_w and n_chunks % NBUF == 0
    n_outer = n_chunks // NBUF

    mesh = plsc.VectorSubcoreMesh(core_axis_name="c", subcore_axis_name="s")

    @functools.partial(
        pl.kernel,
        mesh=mesh,
        out_type=jax.ShapeDtypeStruct((B, EMBED), jnp.float32),
        scratch_types=[
            pltpu.VMEM((b_per_w,), jnp.int32),
            pltpu.VMEM((NBUF, CHUNK, EMBED), jnp.float32),
            [pltpu.SemaphoreType.DMA] * NBUF,
            [pltpu.SemaphoreType.DMA] * NBUF,
        ],
        compiler_params=pltpu.CompilerParams(use_tc_tiling_on_sc=False),
    )
    def k(table_hbm, idx_hbm, out_hbm, idx_v, rows_v, gsems, ssems):
        wid = lax.axis_index("s") * _NC + lax.axis_index("c")
        base = wid * b_per_w

        pltpu.sync_copy(idx_hbm.at[pl.ds(base, b_per_w)], idx_v)

        def start_gather(g, b):
            pltpu.async_copy(
                table_hbm.at[idx_v.at[pl.ds(g * CHUNK, CHUNK)]],
                rows_v.at[b], gsems[b])

        def wait_gather(g, b):
            pltpu.make_async_copy(
                table_hbm.at[idx_v.at[pl.ds(g * CHUNK, CHUNK)]],
                rows_v.at[b], gsems[b]).wait()

        def start_store(g, b):
            pltpu.async_copy(
                rows_v.at[b], out_hbm.at[pl.ds(base + g * CHUNK, CHUNK)],
                ssems[b])

        def wait_store(g, b):
            pltpu.make_async_copy(
                rows_v.at[b], out_hbm.at[pl.ds(base + g * CHUNK, CHUNK)],
                ssems[b]).wait()

        for b in range(NBUF):
            start_gather(b, b)

        def outer(t, carry):
            for b in range(NBUF):
                g = t * NBUF + b
                wait_gather(g, b)
                start_store(g, b)

                @pl.when(t < n_outer - 1)
                def _():
                    wait_store(g, b)
                    start_gather(g + NBUF, b)

            return carry

        lax.fori_loop(0, n_outer, outer, 0)
        for b in range(NBUF):
            wait_store(n_chunks - NBUF + b, b)

    return k(W_embed, flat_idx)


def kernel(x, W_embed):
    flat = x.reshape(-1).astype(jnp.int32)
    out = _embedding_gather(W_embed, flat)
    return out.reshape(x.shape + (EMBED,))
